# Initial kernel scaffold; baseline (speedup 1.0000x reference)
import jax
import jax.numpy as jnp
from jax.experimental import pallas as pl


def kernel(token_ids, tok_table, pos_table):
    raise NotImplementedError("write your pallas kernel here")


# speedup vs baseline: 1.3140x; 1.3140x over previous
import functools
import math

import jax
import jax.numpy as jnp
from jax import lax
from jax.experimental import pallas as pl
from jax.experimental.pallas import tpu as pltpu
from jax.experimental.pallas import tpu_sc as plsc

VOCAB = 100000
D = 128
B = 4
T = 2048
FLAT = B * T
NC, NS, L = 2, 16, 16
NW = NC * NS
BPW = FLAT // NW
SCALE = math.sqrt(D)

_mesh = plsc.VectorSubcoreMesh(core_axis_name="c", subcore_axis_name="s")


@functools.partial(
    pl.kernel,
    mesh=_mesh,
    out_type=jax.ShapeDtypeStruct((FLAT, D), jnp.float32),
    scratch_types=[
        pltpu.VMEM((BPW,), jnp.int32),
        pltpu.VMEM((BPW, D), jnp.float32),
        pltpu.VMEM((BPW, D), jnp.float32),
        pltpu.SemaphoreType.DMA,
        pltpu.SemaphoreType.DMA,
    ],
)
def _embed(idx_hbm, tok_hbm, pos_hbm, out_hbm, idx_v, rows_v, pos_v,
           gsem, psem):
    wid = lax.axis_index("s") * NC + lax.axis_index("c")
    base = wid * BPW
    pos_base = lax.rem(wid, T // BPW) * BPW

    pltpu.sync_copy(idx_hbm.at[pl.ds(base, BPW)], idx_v)
    gather = pltpu.async_copy(tok_hbm.at[idx_v], rows_v, gsem)
    pcopy = pltpu.async_copy(pos_hbm.at[pl.ds(pos_base, BPW)], pos_v, psem)
    gather.wait()
    pcopy.wait()

    def body(r, carry):
        for j in range(D // L):
            sl = pl.ds(j * L, L)
            rows_v[r, sl] = rows_v[r, sl] * SCALE + pos_v[r, sl]
        return carry

    lax.fori_loop(0, BPW, body, 0)

    pltpu.sync_copy(rows_v, out_hbm.at[pl.ds(base, BPW)])


def kernel(token_ids, tok_table, pos_table):
    idx = token_ids.reshape(FLAT).astype(jnp.int32)
    out = _embed(idx, tok_table, pos_table)
    return out.reshape(B, T, D)


# speedup vs baseline: 1.3499x; 1.0273x over previous
import functools
import math

import jax
import jax.numpy as jnp
from jax import lax
from jax.experimental import pallas as pl
from jax.experimental.pallas import tpu as pltpu
from jax.experimental.pallas import tpu_sc as plsc

VOCAB = 100000
D = 128
B = 4
T = 2048
FLAT = B * T
NC, NS, L = 2, 16, 16
NW = NC * NS
BPW = FLAT // NW
NCHUNK = 4
CK = BPW // NCHUNK
SCALE = math.sqrt(D)

_mesh = plsc.VectorSubcoreMesh(core_axis_name="c", subcore_axis_name="s")


@functools.partial(
    pl.kernel,
    mesh=_mesh,
    out_type=jax.ShapeDtypeStruct((FLAT, D), jnp.float32),
    scratch_types=[
        pltpu.VMEM((NCHUNK, CK), jnp.int32),
        pltpu.VMEM((BPW, D), jnp.float32),
        pltpu.VMEM((BPW, D), jnp.float32),
        pltpu.SemaphoreType.DMA,
        pltpu.SemaphoreType.DMA,
        pltpu.SemaphoreType.DMA,
        pltpu.SemaphoreType.DMA,
        pltpu.SemaphoreType.DMA,
    ],
)
def _embed(idx_hbm, tok_hbm, pos_hbm, out_hbm, idx_v, rows_v, pos_v,
           s0, s1, s2, s3, osem):
    wid = lax.axis_index("s") * NC + lax.axis_index("c")
    base = wid * BPW
    pos_base = lax.rem(wid, T // BPW) * BPW
    sems = (s0, s1, s2, s3)

    pltpu.sync_copy(idx_hbm.at[wid], idx_v)

    waits = []
    for k in range(NCHUNK):
        g = pltpu.async_copy(
            tok_hbm.at[idx_v.at[k]], rows_v.at[pl.ds(k * CK, CK)], sems[k])
        p = pltpu.async_copy(
            pos_hbm.at[pl.ds(pos_base + k * CK, CK)],
            pos_v.at[pl.ds(k * CK, CK)], sems[k])
        waits.append((g, p))

    out_waits = []
    for k in range(NCHUNK):
        g, p = waits[k]
        g.wait()
        p.wait()

        def body(r, carry, k=k):
            row = k * CK + r
            for j in range(D // L):
                sl = pl.ds(j * L, L)
                rows_v[row, sl] = rows_v[row, sl] * SCALE + pos_v[row, sl]
            return carry

        lax.fori_loop(0, CK, body, 0)
        out_waits.append(pltpu.async_copy(
            rows_v.at[pl.ds(k * CK, CK)],
            out_hbm.at[pl.ds(base + k * CK, CK)], osem))

    for w in out_waits:
        w.wait()


def kernel(token_ids, tok_table, pos_table):
    idx = token_ids.reshape(NW, NCHUNK, CK).astype(jnp.int32)
    out = _embed(idx, tok_table, pos_table)
    return out.reshape(B, T, D)
